# Initial kernel scaffold; baseline (speedup 1.0000x reference)
import jax
import jax.numpy as jnp
from jax.experimental import pallas as pl


def kernel(f, f_norm, a, rel_pos, edge_attr, edge_index, We1, be1, We12, be12, Wh1, bh1, Wh12, bh12, WA, bA, We2, be2, Wh2, bh2, Wg, bg, Wd, bd, Wu, bu):
    raise NotImplementedError("write your pallas kernel here")


# speedup vs baseline: 2.1768x; 2.1768x over previous
import functools

import jax
import jax.numpy as jnp
from jax import lax
from jax.experimental import pallas as pl
from jax.experimental.pallas import tpu as pltpu
from jax.experimental.pallas import tpu_sc as plsc

N = 10000
E = 320000
L = 6
NC = 2
NS = 16
NW = NC * NS
CH = 128
CPT = 80
EPT = CH * CPT
E_PAD = NW * EPT
N_ROWS_PT = 632
N_PAD = NS * N_ROWS_PT
BE = 2048


def _swish(x):
    return x * jax.nn.sigmoid(x)


def _dot(x, w):
    return jnp.dot(x, w, preferred_element_type=jnp.float32)


def _embed_e_body(x_ref, w1_ref, b1_ref, w2_ref, b2_ref, o_ref):
    t = _swish(_dot(x_ref[...], w1_ref[...]) + b1_ref[...])
    o_ref[...] = _swish(_dot(t, w2_ref[...]) + b2_ref[...])


def _embed_e(xe, w1, b1, w2, b2):
    return pl.pallas_call(
        _embed_e_body,
        grid=(E_PAD // BE,),
        in_specs=[
            pl.BlockSpec((BE, 19), lambda i: (i, 0)),
            pl.BlockSpec((19, 128), lambda i: (0, 0)),
            pl.BlockSpec((1, 128), lambda i: (0, 0)),
            pl.BlockSpec((128, 128), lambda i: (0, 0)),
            pl.BlockSpec((1, 128), lambda i: (0, 0)),
        ],
        out_specs=pl.BlockSpec((BE, 128), lambda i: (i, 0)),
        out_shape=jax.ShapeDtypeStruct((E_PAD, 128), jnp.float32),
    )(xe, w1, b1, w2, b2)


def _emit_tables(h, wgs_ref, wgd_ref, wd_ref, bd_ref, tsrc_ref, tdst_ref):
    tsrc_ref[:, :128] = _dot(h, wgs_ref[...])
    tsrc_ref[:, 128:] = _swish(_dot(h, wd_ref[...]) + bd_ref[...])
    tdst_ref[...] = _dot(h, wgd_ref[...])


def _node0_body(xh_ref, w1_ref, b1_ref, w2_ref, b2_ref,
                wgs_ref, wgd_ref, wd_ref, bd_ref, tsrc_ref, tdst_ref):
    t = _swish(_dot(xh_ref[...], w1_ref[...]) + b1_ref[...])
    h = _swish(_dot(t, w2_ref[...]) + b2_ref[...])
    _emit_tables(h, wgs_ref, wgd_ref, wd_ref, bd_ref, tsrc_ref, tdst_ref)


def _node0(xh, w1, b1, w2, b2, wgs, wgd, wd, bd):
    return pl.pallas_call(
        _node0_body,
        out_shape=[
            jax.ShapeDtypeStruct((N, 256), jnp.float32),
            jax.ShapeDtypeStruct((N, 128), jnp.float32),
        ],
    )(xh, w1, b1, w2, b2, wgs, wgd, wd, bd)


def _noden_body(p_ref, wu_ref, bu_ref,
                wgs_ref, wgd_ref, wd_ref, bd_ref, tsrc_ref, tdst_ref):
    agg = p_ref[0:N, :] + p_ref[N_PAD:N_PAD + N, :]
    h = _swish(_dot(agg, wu_ref[...]) + bu_ref[...])
    _emit_tables(h, wgs_ref, wgd_ref, wd_ref, bd_ref, tsrc_ref, tdst_ref)


def _noden(p, wu, bu, wgs, wgd, wd, bd):
    return pl.pallas_call(
        _noden_body,
        out_shape=[
            jax.ShapeDtypeStruct((N, 256), jnp.float32),
            jax.ShapeDtypeStruct((N, 128), jnp.float32),
        ],
    )(p, wu, bu, wgs, wgd, wd, bd)


def _node_final_body(p_ref, wu_ref, bu_ref, h_ref):
    agg = p_ref[0:N, :] + p_ref[N_PAD:N_PAD + N, :]
    h_ref[...] = _swish(_dot(agg, wu_ref[...]) + bu_ref[...])


def _node_final(p, wu, bu):
    return pl.pallas_call(
        _node_final_body,
        out_shape=jax.ShapeDtypeStruct((N, 128), jnp.float32),
    )(p, wu, bu)


def _edge_body(e_ref, g1_ref, g2_ref, wge_ref, bg_ref, o_ref):
    pre = _dot(e_ref[...], wge_ref[...]) + g1_ref[:, :128] + g2_ref[...] + bg_ref[...]
    o_ref[...] = g1_ref[:, 128:] * _swish(pre)


def _edge(e, g1, g2, wge, bg2d):
    return pl.pallas_call(
        _edge_body,
        grid=(E_PAD // BE,),
        in_specs=[
            pl.BlockSpec((BE, 128), lambda i: (i, 0)),
            pl.BlockSpec((BE, 256), lambda i: (i, 0)),
            pl.BlockSpec((BE, 128), lambda i: (i, 0)),
            pl.BlockSpec((128, 128), lambda i: (0, 0)),
            pl.BlockSpec((1, 128), lambda i: (0, 0)),
        ],
        out_specs=pl.BlockSpec((BE, 128), lambda i: (i, 0)),
        out_shape=jax.ShapeDtypeStruct((E_PAD, 128), jnp.float32),
    )(e, g1, g2, wge, bg2d)


@functools.cache
def _make_sc_gather():
    mesh = plsc.VectorSubcoreMesh(core_axis_name="c", subcore_axis_name="s",
                                  num_cores=NC)

    @functools.partial(
        pl.kernel,
        mesh=mesh,
        out_type=[
            jax.ShapeDtypeStruct((E_PAD, 256), jnp.float32),
            jax.ShapeDtypeStruct((E_PAD, 128), jnp.float32),
        ],
        scratch_types=[
            pltpu.VMEM((CPT, CH), jnp.int32),
            pltpu.VMEM((CPT, CH), jnp.int32),
            pltpu.VMEM((CH, 256), jnp.float32),
            pltpu.VMEM((CH, 128), jnp.float32),
            pltpu.SemaphoreType.DMA,
            pltpu.SemaphoreType.DMA,
        ],
    )
    def sc_gather(tsrc_hbm, tdst_hbm, src_hbm, dstg_hbm, g1_hbm, g2_hbm,
                  src_v, dstg_v, g1_v, g2_v, sem1, sem2):
        wid = lax.axis_index("s") * NC + lax.axis_index("c")
        row0 = wid * CPT
        ci = pltpu.async_copy(src_hbm.at[pl.ds(row0, CPT)], src_v, sem1)
        cj = pltpu.async_copy(dstg_hbm.at[pl.ds(row0, CPT)], dstg_v, sem2)
        ci.wait()
        cj.wait()

        def chunk(i, carry):
            c1 = pltpu.async_copy(tsrc_hbm.at[src_v.at[i]], g1_v, sem1)
            c2 = pltpu.async_copy(tdst_hbm.at[dstg_v.at[i]], g2_v, sem2)
            c1.wait()
            c2.wait()
            base = wid * EPT + i * CH
            c3 = pltpu.async_copy(g1_v, g1_hbm.at[pl.ds(base, CH)], sem1)
            c4 = pltpu.async_copy(g2_v, g2_hbm.at[pl.ds(base, CH)], sem2)
            c3.wait()
            c4.wait()
            return carry

        lax.fori_loop(0, CPT, chunk, 0)

    return sc_gather


@functools.cache
def _make_sc_scatter():
    mesh = plsc.VectorSubcoreMesh(core_axis_name="c", subcore_axis_name="s",
                                  num_cores=NC)

    @functools.partial(
        pl.kernel,
        mesh=mesh,
        out_type=jax.ShapeDtypeStruct((NC * N_PAD, 128), jnp.float32),
        scratch_types=[
            pltpu.VMEM((CPT, CH), jnp.int32),
            pltpu.VMEM((CH, 128), jnp.float32),
            pltpu.VMEM_SHARED((N_PAD, 128), jnp.float32),
            pltpu.SemaphoreType.DMA,
        ],
    )
    def sc_scatter(msg_hbm, dsts_hbm, zeros_hbm, out_hbm, dsts_v, msg_v, acc, sem):
        c = lax.axis_index("c")
        s = lax.axis_index("s")
        wid = s * NC + c
        pltpu.sync_copy(zeros_hbm.at[pl.ds(s * N_ROWS_PT, N_ROWS_PT)],
                        acc.at[pl.ds(s * N_ROWS_PT, N_ROWS_PT)])
        pltpu.async_copy(dsts_hbm.at[pl.ds(wid * CPT, CPT)], dsts_v, sem).wait()
        plsc.subcore_barrier()

        def chunk(i, carry):
            pltpu.sync_copy(msg_hbm.at[pl.ds(wid * EPT + i * CH, CH)], msg_v)
            pltpu.sync_copy(msg_v, acc.at[dsts_v.at[i]], add=True)
            return carry

        lax.fori_loop(0, CPT, chunk, 0)
        plsc.subcore_barrier()
        pltpu.sync_copy(acc.at[pl.ds(s * N_ROWS_PT, N_ROWS_PT)],
                        out_hbm.at[pl.ds(c * N_PAD + s * N_ROWS_PT, N_ROWS_PT)])

    return sc_scatter


def kernel(f, f_norm, a, rel_pos, edge_attr, edge_index, We1, be1, We12, be12,
           Wh1, bh1, Wh12, bh12, WA, bA, We2, be2, Wh2, bh2, Wg, bg, Wd, bd,
           Wu, bu):
    f32 = jnp.float32
    src = edge_index[0].astype(jnp.int32)
    dst = edge_index[1].astype(jnp.int32)
    padz = jnp.zeros((E_PAD - E,), jnp.int32)
    src_p = jnp.concatenate([src, padz]).reshape(NW * CPT, CH)
    dstg_p = jnp.concatenate([dst, padz]).reshape(NW * CPT, CH)
    dsts_p = jnp.concatenate(
        [dst, jnp.full((E_PAD - E,), N, jnp.int32)]).reshape(NW * CPT, CH)

    xe = jnp.pad(jnp.concatenate([rel_pos, edge_attr], axis=1),
                 ((0, E_PAD - E), (0, 0)))
    w1e = jnp.zeros((19, 128), f32).at[0:3, 0:64].set(We1).at[3:19, 64:128].set(We12)
    b1e = jnp.concatenate([be1, be12]).reshape(1, 128)
    xh = jnp.concatenate([f, f_norm, a], axis=1)
    w1h = (jnp.zeros((20, 128), f32).at[0:3, 0:64].set(Wh1)
           .at[3:19, 64:124].set(Wh12).at[19:20, 124:128].set(WA))
    b1h = jnp.concatenate([bh1, bh12, bA]).reshape(1, 128)
    zeros_nt = jnp.zeros((N_PAD, 128), f32)

    e = _embed_e(xe, w1e, b1e, We2, be2.reshape(1, 128))
    tsrc, tdst = _node0(xh, w1h, b1h, Wh2, bh2.reshape(1, 128),
                        Wg[0, 128:256], Wg[0, 256:384], Wd[0], bd[0].reshape(1, 128))
    sc_gather = _make_sc_gather()
    sc_scatter = _make_sc_scatter()
    for l in range(L):
        g1, g2 = sc_gather(tsrc, tdst, src_p, dstg_p)
        msg = _edge(e, g1, g2, Wg[l, :128], bg[l].reshape(1, 128))
        p = sc_scatter(msg, dsts_p, zeros_nt)
        if l < L - 1:
            tsrc, tdst = _noden(p, Wu[l], bu[l].reshape(1, 128),
                                Wg[l + 1, 128:256], Wg[l + 1, 256:384],
                                Wd[l + 1], bd[l + 1].reshape(1, 128))
        else:
            h = _node_final(p, Wu[l], bu[l].reshape(1, 128))
    return h
